# baseline (speedup 1.0000x reference)
import functools

import jax
import jax.numpy as jnp
from jax import lax
from jax.experimental import pallas as pl
from jax.experimental.pallas import tpu as pltpu
from jax.experimental.pallas import tpu_sc as plsc

N_BINS = 15
N_ROWS = 1_000_000
N_CLS = 100

TC_BLOCK = 4000
TC_GRID = N_ROWS // TC_BLOCK

LANES = 16
N_TILES = 16
UNROLL = 4
VREGS_PER_TILE = 977 * UNROLL
PER_TILE = VREGS_PER_TILE * LANES
PAD_N = PER_TILE * N_TILES


def _tc_body(x_ref, lab_ref, out_ref):
    xt = x_ref[0].T
    lab = lab_ref[0, 0]
    m = jnp.max(xt, axis=0)
    e = jnp.exp(xt - m[None, :])
    s = jnp.sum(e, axis=0)
    row = lax.broadcasted_iota(jnp.int32, xt.shape, 0)
    hit = jnp.where((row == lab[None, :]) & (xt == m[None, :]), 1.0, 0.0)
    cnt = jnp.sum(hit, axis=0)
    conf = 1.0 / s
    out_ref[0, 0] = jnp.where(cnt > 0.5, -conf, conf)


def _tc_stage(logits, labels):
    return pl.pallas_call(
        _tc_body,
        grid=(TC_GRID,),
        in_specs=[
            pl.BlockSpec((1, TC_BLOCK, N_CLS), lambda i: (i, 0, 0)),
            pl.BlockSpec((1, 1, TC_BLOCK), lambda i: (i, 0, 0)),
        ],
        out_specs=pl.BlockSpec((1, 1, TC_BLOCK), lambda i: (i, 0, 0)),
        out_shape=jax.ShapeDtypeStruct((TC_GRID, 1, TC_BLOCK), jnp.float32),
    )(
        logits.reshape(TC_GRID, TC_BLOCK, N_CLS),
        labels.astype(jnp.int32).reshape(TC_GRID, 1, TC_BLOCK),
    )


def _sc_body(packed_hbm, out_hbm, buf_v, cnt_v, sconf_v, sacc_v,
             outbuf_v, cnt_sh, sconf_sh, sacc_sh):
    tid = lax.axis_index("s")
    base = tid * PER_TILE
    pltpu.sync_copy(packed_hbm.at[pl.ds(base, PER_TILE)], buf_v)

    lane = lax.iota(jnp.int32, LANES)
    lane16 = lane * 16
    zeros = jnp.zeros((LANES,), jnp.float32)
    ones = jnp.ones((LANES,), jnp.float32)
    for r in range(16):
        cnt_v[pl.ds(r * LANES, LANES)] = zeros
        sconf_v[pl.ds(r * LANES, LANES)] = zeros
        sacc_v[pl.ds(r * LANES, LANES)] = zeros

    def body(k, carry):
        off = k * (UNROLL * LANES)
        for u in range(UNROLL):
            v = buf_v[pl.ds(off + u * LANES, LANES)]
            conf = jnp.abs(v)
            is_corr = v < 0.0
            t = conf * jnp.float32(N_BINS)
            b = t.astype(jnp.int32)
            b = b + jnp.where(t > b.astype(jnp.float32), 1, 0)
            b = jnp.minimum(b, N_BINS)
            slot = lane16 + b
            plsc.addupdate_scatter(cnt_v, [slot], ones)
            plsc.addupdate_scatter(sconf_v, [slot], conf)
            plsc.addupdate_scatter(sacc_v, [slot],
                                   jnp.where(is_corr, 1.0, 0.0).astype(jnp.float32))
        return carry

    lax.fori_loop(0, VREGS_PER_TILE // UNROLL, body, 0)

    pltpu.sync_copy(cnt_v, cnt_sh.at[tid])
    pltpu.sync_copy(sconf_v, sconf_sh.at[tid])
    pltpu.sync_copy(sacc_v, sacc_sh.at[tid])

    plsc.subcore_barrier()

    @pl.when(tid == 0)
    def _finalize():
        nslots = N_TILES * 256
        for t in range(N_TILES):
            pltpu.sync_copy(cnt_sh.at[t], buf_v.at[pl.ds(t * 256, 256)])
            pltpu.sync_copy(sconf_sh.at[t],
                            buf_v.at[pl.ds(nslots + t * 256, 256)])
            pltpu.sync_copy(sacc_sh.at[t],
                            buf_v.at[pl.ds(2 * nslots + t * 256, 256)])
        cnt = zeros
        sc = zeros
        sa = zeros
        for r in range(nslots // LANES):
            cnt = cnt + buf_v[pl.ds(r * LANES, LANES)]
            sc = sc + buf_v[pl.ds(nslots + r * LANES, LANES)]
            sa = sa + buf_v[pl.ds(2 * nslots + r * LANES, LANES)]
        safe = jnp.maximum(cnt, 1.0)
        contrib = jnp.abs(sc / safe - sa / safe) * (cnt * jnp.float32(1.0 / N_ROWS))
        valid = (cnt > 0.0) & (lane > 0)
        contrib = jnp.where(valid, contrib, 0.0)
        outbuf_v[...] = jnp.broadcast_to(jnp.sum(contrib), (LANES,))
        pltpu.sync_copy(outbuf_v, out_hbm)


@functools.cache
def _sc_histogram_fn():
    return functools.partial(
        pl.kernel,
        out_type=jax.ShapeDtypeStruct((LANES,), jnp.float32),
        mesh=plsc.VectorSubcoreMesh(
            core_axis_name="c", subcore_axis_name="s", num_cores=1),
        compiler_params=pltpu.CompilerParams(needs_layout_passes=False),
        scratch_types=[
            pltpu.VMEM((PER_TILE,), jnp.float32),
            pltpu.VMEM((256,), jnp.float32),
            pltpu.VMEM((256,), jnp.float32),
            pltpu.VMEM((256,), jnp.float32),
            pltpu.VMEM((LANES,), jnp.float32),
            pltpu.VMEM_SHARED((N_TILES, 256), jnp.float32),
            pltpu.VMEM_SHARED((N_TILES, 256), jnp.float32),
            pltpu.VMEM_SHARED((N_TILES, 256), jnp.float32),
        ],
    )(_sc_body)


@jax.jit
def kernel(logits, labels):
    packed = _tc_stage(logits, labels).reshape(N_ROWS)
    padded = jnp.concatenate(
        [packed, jnp.zeros((PAD_N - N_ROWS,), jnp.float32)])
    ece_vec = _sc_histogram_fn()(padded)
    return ece_vec[0:1]
